# baseline (speedup 1.0000x reference)
import jax
import jax.numpy as jnp
from jax.experimental import pallas as pl
from jax.experimental.pallas import tpu as pltpu

_C = 512
_NBUF = 4


def _router_body(xa_ref, xh_hbm, w1_ref, b1_ref, w2_ref, b2_ref, o_ref,
                 xbuf, sems):
    i = pl.program_id(0)
    n_steps = pl.num_programs(0)
    half = n_steps * _C

    def _copy(j, slot):
        return pltpu.make_async_copy(
            xh_hbm.at[pl.ds(half + j * _C, _C), :], xbuf.at[slot],
            sems.at[slot])

    @pl.when(i == 0)
    def _():
        for j in range(_NBUF):
            _copy(j, j).start()

    def _block(xblk, row0):
        h = jnp.dot(xblk, w1_ref[...], preferred_element_type=jnp.float32)
        h = jnp.maximum(h + b1_ref[...], 0.0)
        logits = jnp.dot(h, w2_ref[...], preferred_element_type=jnp.float32)
        logits = logits + b2_ref[...]
        m = jnp.max(logits, axis=-1, keepdims=True)
        e = jnp.exp(logits - m)
        o_ref[pl.ds(row0, _C), :] = e / jnp.sum(e, axis=-1, keepdims=True)

    _block(xa_ref[...], i * _C)

    slot = jax.lax.rem(i, _NBUF)
    _copy(i, slot).wait()
    _block(xbuf[slot], half + i * _C)

    @pl.when(i + _NBUF < n_steps)
    def _():
        _copy(i + _NBUF, slot).start()


def kernel(x, W1, b1, W2, b2):
    M, K = x.shape
    H = W1.shape[1]
    E = W2.shape[1]
    n_steps = M // (2 * _C)

    b1r = b1.reshape(1, H)
    b2r = b2.reshape(1, E)

    return pl.pallas_call(
        _router_body,
        grid=(n_steps,),
        in_specs=[
            pl.BlockSpec((_C, K), lambda i: (i, 0)),
            pl.BlockSpec(memory_space=pltpu.HBM),
            pl.BlockSpec(memory_space=pltpu.VMEM),
            pl.BlockSpec(memory_space=pltpu.VMEM),
            pl.BlockSpec(memory_space=pltpu.VMEM),
            pl.BlockSpec(memory_space=pltpu.VMEM),
        ],
        out_specs=pl.BlockSpec(memory_space=pltpu.VMEM),
        out_shape=jax.ShapeDtypeStruct((M, E), jnp.float32),
        scratch_shapes=[
            pltpu.VMEM((_NBUF, _C, K), jnp.float32),
            pltpu.SemaphoreType.DMA((_NBUF,)),
        ],
        compiler_params=pltpu.CompilerParams(
            dimension_semantics=("arbitrary",),
            skip_device_barrier=True,
            disable_semaphore_checks=True,
            disable_bounds_checks=True,
        ),
    )(x, x, W1, b1r, W2, b2r)


# speedup vs baseline: 1.0395x; 1.0395x over previous
import jax
import jax.numpy as jnp
from jax.experimental import pallas as pl
from jax.experimental.pallas import tpu as pltpu

_C = 256
_NBUF = 12


def _router_body(x_hbm, w1_ref, b1_ref, w2_ref, b2_ref, o_ref, xbuf, sems):
    i = pl.program_id(0)
    n_steps = pl.num_programs(0)

    def _copy(j, slot):
        return pltpu.make_async_copy(
            x_hbm.at[pl.ds(j * _C, _C), :], xbuf.at[slot], sems.at[slot])

    @pl.when(i == 0)
    def _():
        for j in range(_NBUF):
            _copy(j, j).start()

    slot = jax.lax.rem(i, _NBUF)
    _copy(i, slot).wait()
    h = jnp.dot(xbuf[slot], w1_ref[...], preferred_element_type=jnp.float32)
    h = jnp.maximum(h + b1_ref[...], 0.0)
    logits = jnp.dot(h, w2_ref[...], preferred_element_type=jnp.float32)
    logits = logits + b2_ref[...]
    m = jnp.max(logits, axis=-1, keepdims=True)
    e = jnp.exp(logits - m)
    o_ref[...] = e / jnp.sum(e, axis=-1, keepdims=True)

    @pl.when(i + _NBUF < n_steps)
    def _():
        _copy(i + _NBUF, slot).start()


def kernel(x, W1, b1, W2, b2):
    M, K = x.shape
    H = W1.shape[1]
    E = W2.shape[1]
    n_steps = M // _C

    b1r = b1.reshape(1, H)
    b2r = b2.reshape(1, E)

    return pl.pallas_call(
        _router_body,
        grid=(n_steps,),
        in_specs=[
            pl.BlockSpec(memory_space=pltpu.HBM),
            pl.BlockSpec(memory_space=pltpu.VMEM),
            pl.BlockSpec(memory_space=pltpu.VMEM),
            pl.BlockSpec(memory_space=pltpu.VMEM),
            pl.BlockSpec(memory_space=pltpu.VMEM),
        ],
        out_specs=pl.BlockSpec((_C, E), lambda i: (i, 0)),
        out_shape=jax.ShapeDtypeStruct((M, E), jnp.float32),
        scratch_shapes=[
            pltpu.VMEM((_NBUF, _C, K), jnp.float32),
            pltpu.SemaphoreType.DMA((_NBUF,)),
        ],
        compiler_params=pltpu.CompilerParams(
            dimension_semantics=("arbitrary",),
            skip_device_barrier=True,
            disable_semaphore_checks=True,
            disable_bounds_checks=True,
            vmem_limit_bytes=60 * 1024 * 1024,
        ),
    )(x, W1, b1r, W2, b2r)


# speedup vs baseline: 1.2157x; 1.1695x over previous
import jax
import jax.numpy as jnp
from jax.experimental import pallas as pl
from jax.experimental.pallas import tpu as pltpu

_BM = 1024


def _router_block(x_ref, w1_ref, b1_ref, w2_ref, b2_ref, o_ref):
    h = jnp.dot(x_ref[...], w1_ref[...], preferred_element_type=jnp.float32)
    h = jnp.maximum(h + b1_ref[...], 0.0)
    logits = jnp.dot(h, w2_ref[...], preferred_element_type=jnp.float32)
    logits = logits + b2_ref[...]
    m = jnp.max(logits, axis=-1, keepdims=True)
    e = jnp.exp(logits - m)
    o_ref[...] = e / jnp.sum(e, axis=-1, keepdims=True)


def kernel(x, W1, b1, W2, b2):
    M, K = x.shape
    H = W1.shape[1]
    E = W2.shape[1]
    grid = (M // _BM,)

    b1r = b1.reshape(1, H)
    b2r = b2.reshape(1, E)

    return pl.pallas_call(
        _router_block,
        grid=grid,
        in_specs=[
            pl.BlockSpec((_BM, K), lambda i: (i, 0)),
            pl.BlockSpec((K, H), lambda i: (0, 0)),
            pl.BlockSpec((1, H), lambda i: (0, 0)),
            pl.BlockSpec((H, E), lambda i: (0, 0)),
            pl.BlockSpec((1, E), lambda i: (0, 0)),
        ],
        out_specs=pl.BlockSpec((_BM, E), lambda i: (i, 0)),
        out_shape=jax.ShapeDtypeStruct((M, E), jnp.float32),
        compiler_params=pltpu.CompilerParams(
            dimension_semantics=("parallel",),
            skip_device_barrier=True,
            disable_semaphore_checks=True,
            disable_bounds_checks=True,
        ),
    )(x, W1, b1r, W2, b2r)


# speedup vs baseline: 1.2494x; 1.0277x over previous
import jax
import jax.numpy as jnp
from jax.experimental import pallas as pl
from jax.experimental.pallas import tpu as pltpu

_BM = 1024


def _router_block(x_ref, w1_ref, b1_ref, w2_ref, b2_ref, o_ref):
    h = jnp.dot(x_ref[...], w1_ref[...], preferred_element_type=jnp.float32)
    h = jnp.maximum(h + b1_ref[...], 0.0)
    logits = jnp.dot(h, w2_ref[...], preferred_element_type=jnp.float32)
    logits = logits + b2_ref[...]
    m = jnp.max(logits, axis=-1, keepdims=True)
    e = jnp.exp(logits - m)
    o_ref[...] = e / jnp.sum(e, axis=-1, keepdims=True)


def kernel(x, W1, b1, W2, b2):
    M, K = x.shape
    H = W1.shape[1]
    E = W2.shape[1]
    grid = (M // _BM,)

    b1r = b1.reshape(1, H)
    b2r = b2.reshape(1, E)

    return pl.pallas_call(
        _router_block,
        grid=grid,
        in_specs=[
            pl.BlockSpec((_BM, K), lambda i: (i, 0)),
            pl.BlockSpec((K, H), lambda i: (0, 0)),
            pl.BlockSpec((1, H), lambda i: (0, 0)),
            pl.BlockSpec((H, E), lambda i: (0, 0)),
            pl.BlockSpec((1, E), lambda i: (0, 0)),
        ],
        out_specs=pl.BlockSpec((_BM, E), lambda i: (i, 0)),
        out_shape=jax.ShapeDtypeStruct((M, E), jnp.float32),
        compiler_params=pltpu.CompilerParams(
            dimension_semantics=("parallel",),
        ),
    )(x, W1, b1r, W2, b2r)
